# Initial kernel scaffold; baseline (speedup 1.0000x reference)
import jax
import jax.numpy as jnp
from jax.experimental import pallas as pl


def kernel(vertices, dir0, w1, b1, dir1, w2, b2, dir2, w3, b3, dir3, w4, b4, dir4, cls_w1, cls_b1, cls_gamma, cls_beta, cls_w2, cls_b2, proj_w1, proj_b1, proj_gamma, proj_beta, proj_w2, proj_b2):
    raise NotImplementedError("write your pallas kernel here")


# speedup vs baseline: 1.2298x; 1.2298x over previous
import functools
import math

import jax
import jax.numpy as jnp
from jax import lax
from jax.experimental import pallas as pl
from jax.experimental.pallas import tpu as pltpu

BS = 2
V = 2048
NB = 16
K1 = NB + 1
RB = 256


def _topk_body(vb_ref, va_ref, idx_ref):
    vb = vb_ref[0]
    va = va_ref[0]
    inner = lax.dot_general(vb, va, (((1,), (1,)), ((), ())),
                            precision=lax.Precision.DEFAULT)
    qa = jnp.sum(va * va, axis=1)
    qb = jnp.sum(vb * vb, axis=1)
    d = (-2.0 * inner + qa[None, :]) + qb[:, None]
    cols = lax.broadcasted_iota(jnp.int32, (RB, V), 1)
    picks = []
    for _ in range(K1):
        m = jnp.min(d, axis=1, keepdims=True)
        cand = jnp.where(d == m, cols, V)
        jj = jnp.min(cand, axis=1, keepdims=True)
        picks.append(jj)
        d = jnp.where(cols == jj, jnp.inf, d)
    idx_ref[0] = jnp.concatenate(picks, axis=1)


def _knn_topk(vertices):
    return pl.pallas_call(
        _topk_body,
        grid=(BS, V // RB),
        in_specs=[
            pl.BlockSpec((1, RB, 3), lambda b, r: (b, r, 0)),
            pl.BlockSpec((1, V, 3), lambda b, r: (b, 0, 0)),
        ],
        out_specs=pl.BlockSpec((1, RB, K1), lambda b, r: (b, r, 0)),
        out_shape=jax.ShapeDtypeStruct((BS, V, K1), jnp.int32),
    )(vertices, vertices)


def _normalize(x, axis, eps=1e-12):
    n = jnp.sqrt(jnp.sum(x * x, axis=axis, keepdims=True))
    return x / jnp.maximum(n, eps)


def _indexing_neighbor(tensor, index):
    bs = tensor.shape[0]
    return tensor[jnp.arange(bs)[:, None, None], index]


def _gndn(vertices, neighbor_index):
    neighbors = _indexing_neighbor(vertices, neighbor_index)
    d = neighbors - vertices[:, :, None, :]
    return _normalize(d, axis=-1)


def _conv_surface(neighbor_index, vertices, directions, kernel_num):
    bs, v, n = neighbor_index.shape
    ndn = _gndn(vertices, neighbor_index)
    sdn = _normalize(directions, axis=0)
    theta = jax.nn.relu(ndn @ sdn)
    theta = theta.reshape(bs, v, n, 1, kernel_num)
    return jnp.sum(jnp.max(theta, axis=2), axis=2)


def _conv_layer(neighbor_index, vertices, feature_map, weights, bias,
                directions, out_channel):
    bs, v, n = neighbor_index.shape
    ndn = _gndn(vertices, neighbor_index)
    sdn = _normalize(directions, axis=0)
    theta = jax.nn.relu(ndn @ sdn).reshape(bs, v, n, -1)
    feature_out = feature_map @ weights + bias
    feature_center = feature_out[:, :, :out_channel]
    feature_support = _indexing_neighbor(feature_out[:, :, out_channel:],
                                         neighbor_index)
    act = (theta * feature_support).reshape(bs, v, n, 1, out_channel)
    act = jnp.sum(jnp.max(act, axis=2), axis=2)
    return feature_center + act


def _bn_eval(x, gamma, beta, eps=1e-5):
    return x / jnp.sqrt(1.0 + eps) * gamma + beta


def kernel(vertices, dir0, w1, b1, dir1, w2, b2, dir2, w3, b3, dir3, w4, b4,
           dir4, cls_w1, cls_b1, cls_gamma, cls_beta, cls_w2, cls_b2,
           proj_w1, proj_b1, proj_gamma, proj_beta, proj_w2, proj_b2):
    idx17 = _knn_topk(vertices)
    neighbor_index = idx17[:, :, 1:]
    fm0 = jax.nn.relu(_conv_surface(neighbor_index, vertices, dir0, 32))
    fm1 = jax.nn.relu(_conv_layer(neighbor_index, vertices, fm0, w1, b1, dir1, 64))
    fm2 = jax.nn.relu(_conv_layer(neighbor_index, vertices, fm1, w2, b2, dir2, 128))
    fm3 = jax.nn.relu(_conv_layer(neighbor_index, vertices, fm2, w3, b3, dir3, 256))
    fm4 = _conv_layer(neighbor_index, vertices, fm3, w4, b4, dir4, 1024)
    feature_global = jnp.max(fm4, axis=1)
    h = jax.nn.relu(_bn_eval(feature_global @ cls_w1 + cls_b1, cls_gamma, cls_beta))
    class_output = h @ cls_w2 + cls_b2
    p = jax.nn.relu(_bn_eval(feature_global @ proj_w1 + proj_b1, proj_gamma, proj_beta))
    hidden = p @ proj_w2 + proj_b2
    hidden = _normalize(hidden, axis=1)
    return (feature_global, class_output, hidden)


# speedup vs baseline: 5.5243x; 4.4921x over previous
import functools
import math

import jax
import jax.numpy as jnp
from jax import lax
from jax.experimental import pallas as pl
from jax.experimental.pallas import tpu as pltpu
from jax.experimental.pallas import tpu_sc as plsc

BS = 2
V = 2048
GV = BS * V
NB = 16
K1 = NB + 1
RB = 256

NC, NS, L = 2, 16, 16
NW = NC * NS
VID = GV // NW

_SC_MESH = dict(core_axis_name="c", subcore_axis_name="s")
_SC_PARAMS = pltpu.CompilerParams(needs_layout_passes=False,
                                  use_tc_tiling_on_sc=False)


def _topk_body(vb_ref, va_ref, idx_ref):
    vb = vb_ref[0]
    va = va_ref[0]
    inner = lax.dot_general(vb, va, (((1,), (1,)), ((), ())),
                            precision=lax.Precision.DEFAULT)
    qa = jnp.sum(va * va, axis=1)
    qb = jnp.sum(vb * vb, axis=1)
    d = (-2.0 * inner + qa[None, :]) + qb[:, None]
    cols = lax.broadcasted_iota(jnp.int32, (RB, V), 1)
    picks = []
    for _ in range(K1):
        m = jnp.min(d, axis=1, keepdims=True)
        cand = jnp.where(d == m, cols, V)
        jj = jnp.min(cand, axis=1, keepdims=True)
        picks.append(jj)
        d = jnp.where(cols == jj, jnp.inf, d)
    idx_ref[0] = jnp.concatenate(picks, axis=1)


def _knn_topk(vertices):
    return pl.pallas_call(
        _topk_body,
        grid=(BS, V // RB),
        in_specs=[
            pl.BlockSpec((1, RB, 3), lambda b, r: (b, r, 0)),
            pl.BlockSpec((1, V, 3), lambda b, r: (b, 0, 0)),
        ],
        out_specs=pl.BlockSpec((1, RB, K1), lambda b, r: (b, r, 0)),
        out_shape=jax.ShapeDtypeStruct((BS, V, K1), jnp.int32),
    )(vertices, vertices)


def _rsqrt_nr(s):
    i = lax.bitcast_convert_type(s, jnp.int32)
    y = lax.bitcast_convert_type(
        jnp.int32(0x5F3759DF) - lax.shift_right_arithmetic(i, 1), jnp.float32)
    for _ in range(4):
        y = y * (1.5 - 0.5 * s * y * y)
    return y


def _inv_norm(s):
    return jnp.minimum(_rsqrt_nr(s), jnp.float32(1e12))


def _wid():
    return lax.axis_index("s") * NC + lax.axis_index("c")


def _sc_ndn_body(vx_h, vy_h, vz_h, gidx_h, d0_h,
                 dxn_h, dyn_h, dzn_h, fm0_h,
                 vxv, vyv, vzv, gidxv, d0v, dxnv, dynv, dznv, fm0v):
    wid = _wid()
    base = wid * VID
    pltpu.sync_copy(vx_h, vxv)
    pltpu.sync_copy(vy_h, vyv)
    pltpu.sync_copy(vz_h, vzv)
    pltpu.sync_copy(gidx_h.at[pl.ds(base * NB, VID * NB)], gidxv)
    pltpu.sync_copy(d0_h, d0v)

    sdn = []
    for c in range(2):
        sx = d0v[pl.ds(c * L, L)]
        sy = d0v[pl.ds(32 + c * L, L)]
        sz = d0v[pl.ds(64 + c * L, L)]
        r = _inv_norm(sx * sx + sy * sy + sz * sz)
        sdn.append((sx * r, sy * r, sz * r))

    def body(i, _):
        gv = gidxv[pl.ds(i * NB, NB)]
        ic = jnp.full((L,), base + i, jnp.int32)
        nx = plsc.load_gather(vxv, [gv])
        ny = plsc.load_gather(vyv, [gv])
        nz = plsc.load_gather(vzv, [gv])
        dx = nx - plsc.load_gather(vxv, [ic])
        dy = ny - plsc.load_gather(vyv, [ic])
        dz = nz - plsc.load_gather(vzv, [ic])
        r = _inv_norm(dx * dx + dy * dy + dz * dz)
        dxn = dx * r
        dyn = dy * r
        dzn = dz * r
        dxnv[pl.ds(i * NB, NB)] = dxn
        dynv[pl.ds(i * NB, NB)] = dyn
        dznv[pl.ds(i * NB, NB)] = dzn
        scal = [(dxn[n], dyn[n], dzn[n]) for n in range(NB)]
        for c in range(2):
            sx, sy, sz = sdn[c]
            acc = jnp.zeros((L,), jnp.float32)
            for n in range(NB):
                a, b_, c_ = scal[n]
                th = jnp.maximum(a * sx + b_ * sy + c_ * sz, 0.0)
                acc = jnp.maximum(acc, th)
            fm0v[pl.ds(i * 32 + c * L, L)] = acc
        return 0

    lax.fori_loop(0, VID, body, 0)

    pltpu.sync_copy(dxnv, dxn_h.at[pl.ds(base * NB, VID * NB)])
    pltpu.sync_copy(dynv, dyn_h.at[pl.ds(base * NB, VID * NB)])
    pltpu.sync_copy(dznv, dzn_h.at[pl.ds(base * NB, VID * NB)])
    pltpu.sync_copy(fm0v, fm0_h.at[pl.ds(base * 32, VID * 32)])


def _sc_ndn_fm0(vx, vy, vz, gidx, d0f):
    f = pl.kernel(
        _sc_ndn_body,
        out_type=(
            jax.ShapeDtypeStruct((GV * NB,), jnp.float32),
            jax.ShapeDtypeStruct((GV * NB,), jnp.float32),
            jax.ShapeDtypeStruct((GV * NB,), jnp.float32),
            jax.ShapeDtypeStruct((GV * 32,), jnp.float32),
        ),
        mesh=plsc.VectorSubcoreMesh(**_SC_MESH),
        compiler_params=_SC_PARAMS,
        scratch_types=[
            pltpu.VMEM((GV,), jnp.float32),
            pltpu.VMEM((GV,), jnp.float32),
            pltpu.VMEM((GV,), jnp.float32),
            pltpu.VMEM((VID * NB,), jnp.int32),
            pltpu.VMEM((96,), jnp.float32),
            pltpu.VMEM((VID * NB,), jnp.float32),
            pltpu.VMEM((VID * NB,), jnp.float32),
            pltpu.VMEM((VID * NB,), jnp.float32),
            pltpu.VMEM((VID * 32,), jnp.float32),
        ],
    )
    return f(vx, vy, vz, gidx, d0f)


def _sc_conv_body(co, gidx_h, dxn_h, dyn_h, dzn_h, f_h, dl_h, act_h,
                  gidxv, dxnv, dynv, dznv, dlv, snv, fbuf, actv, sem):
    wid = _wid()
    base = wid * VID
    pltpu.sync_copy(gidx_h.at[pl.ds(base * NB, VID * NB)], gidxv)
    pltpu.sync_copy(dxn_h.at[pl.ds(base * NB, VID * NB)], dxnv)
    pltpu.sync_copy(dyn_h.at[pl.ds(base * NB, VID * NB)], dynv)
    pltpu.sync_copy(dzn_h.at[pl.ds(base * NB, VID * NB)], dznv)
    pltpu.sync_copy(dl_h, dlv)

    nch = co // L

    def nrm(k, _):
        sx = dlv[pl.ds(k * L, L)]
        sy = dlv[pl.ds(co + k * L, L)]
        sz = dlv[pl.ds(2 * co + k * L, L)]
        r = _inv_norm(sx * sx + sy * sy + sz * sz)
        snv[pl.ds(k * L, L)] = sx * r
        snv[pl.ds(co + k * L, L)] = sy * r
        snv[pl.ds(2 * co + k * L, L)] = sz * r
        return 0

    lax.fori_loop(0, nch, nrm, 0)

    def body(i, _):
        gv = gidxv[pl.ds(i * NB, NB)]
        pltpu.async_copy(f_h.at[gv], fbuf, sem).wait()
        dxv = dxnv[pl.ds(i * NB, NB)]
        dyv = dynv[pl.ds(i * NB, NB)]
        dzv = dznv[pl.ds(i * NB, NB)]
        scal = [(dxv[n], dyv[n], dzv[n]) for n in range(NB)]

        def chunk(k, _):
            sx = snv[pl.ds(k * L, L)]
            sy = snv[pl.ds(co + k * L, L)]
            sz = snv[pl.ds(2 * co + k * L, L)]
            a, b_, c_ = scal[0]
            th = jnp.maximum(a * sx + b_ * sy + c_ * sz, 0.0)
            acc = th * fbuf[0, pl.ds(k * L, L)]
            for n in range(1, NB):
                a, b_, c_ = scal[n]
                th = jnp.maximum(a * sx + b_ * sy + c_ * sz, 0.0)
                acc = jnp.maximum(acc, th * fbuf[n, pl.ds(k * L, L)])
            actv[pl.ds(k * L, L)] = acc
            return 0

        lax.fori_loop(0, nch, chunk, 0)
        pltpu.sync_copy(actv, act_h.at[pl.ds((base + i) * co, co)])
        return 0

    lax.fori_loop(0, VID, body, 0)


def _sc_conv(gidx, dxn, dyn, dzn, ftab, dlf, co):
    f = pl.kernel(
        functools.partial(_sc_conv_body, co),
        out_type=jax.ShapeDtypeStruct((GV * co,), jnp.float32),
        mesh=plsc.VectorSubcoreMesh(**_SC_MESH),
        compiler_params=_SC_PARAMS,
        scratch_types=[
            pltpu.VMEM((VID * NB,), jnp.int32),
            pltpu.VMEM((VID * NB,), jnp.float32),
            pltpu.VMEM((VID * NB,), jnp.float32),
            pltpu.VMEM((VID * NB,), jnp.float32),
            pltpu.VMEM((3 * co,), jnp.float32),
            pltpu.VMEM((3 * co,), jnp.float32),
            pltpu.VMEM((NB, co), jnp.float32),
            pltpu.VMEM((co,), jnp.float32),
            pltpu.SemaphoreType.DMA,
        ],
    )
    return f(gidx, dxn, dyn, dzn, ftab, dlf)


def _mm_body(co, has_act, *refs):
    if has_act:
        c_ref, a_ref, w_ref, b_ref, cout_ref, f_ref = refs
        x = jnp.maximum(c_ref[...] + a_ref[...], 0.0)
    else:
        x_ref, w_ref, b_ref, cout_ref, f_ref = refs
        x = x_ref[...]
    out = lax.dot_general(x, w_ref[...], (((1,), (0,)), ((), ())),
                          precision=lax.Precision.DEFAULT) + b_ref[...]
    cout_ref[...] = out[:, :co]
    f_ref[...] = out[:, co:]


def _tc_mm(x_or_center, act, w, b, co):
    ci = w.shape[0]
    has_act = act is not None
    ins = ([x_or_center, act] if has_act else [x_or_center])
    ins += [w, b.reshape(1, 2 * co)]
    in_specs = [pl.BlockSpec((RB, ci), lambda r: (r, 0))
                for _ in range(2 if has_act else 1)]
    in_specs += [pl.BlockSpec((ci, 2 * co), lambda r: (0, 0)),
                 pl.BlockSpec((1, 2 * co), lambda r: (0, 0))]
    return pl.pallas_call(
        functools.partial(_mm_body, co, has_act),
        grid=(GV // RB,),
        in_specs=in_specs,
        out_specs=[pl.BlockSpec((RB, co), lambda r: (r, 0)),
                   pl.BlockSpec((RB, co), lambda r: (r, 0))],
        out_shape=[jax.ShapeDtypeStruct((GV, co), jnp.float32),
                   jax.ShapeDtypeStruct((GV, co), jnp.float32)],
    )(*ins)


def _maxpool_body(c_ref, a_ref, o_ref):
    r = pl.program_id(0)
    blk = c_ref[...] + a_ref[...]
    m = jnp.max(blk, axis=0, keepdims=True)

    @pl.when(r % (V // RB) == 0)
    def _():
        o_ref[0] = m

    @pl.when(r % (V // RB) != 0)
    def _():
        o_ref[0] = jnp.maximum(o_ref[0], m)


def _tc_maxpool(center, act):
    out = pl.pallas_call(
        _maxpool_body,
        grid=(GV // RB,),
        in_specs=[pl.BlockSpec((RB, 1024), lambda r: (r, 0)),
                  pl.BlockSpec((RB, 1024), lambda r: (r, 0))],
        out_specs=pl.BlockSpec((1, 1, 1024), lambda r: (r // (V // RB), 0, 0)),
        out_shape=jax.ShapeDtypeStruct((BS, 1, 1024), jnp.float32),
    )(center, act)
    return out.reshape(BS, 1024)


def _heads_body(fg_ref, cw1, cb1, cg, cbeta, cw2, cb2,
                pw1, pb1, pg, pbeta, pw2, pb2, cls_ref, hid_ref):
    fg = fg_ref[...]
    bnf = 1.0 / jnp.sqrt(jnp.float32(1.0 + 1e-5))
    dot = functools.partial(lax.dot_general,
                            dimension_numbers=(((1,), (0,)), ((), ())),
                            precision=lax.Precision.DEFAULT)
    h = dot(fg, cw1[...]) + cb1[...]
    h = jnp.maximum(h * bnf * cg[...] + cbeta[...], 0.0)
    cls_ref[...] = dot(h, cw2[...]) + cb2[...]
    p = dot(fg, pw1[...]) + pb1[...]
    p = jnp.maximum(p * bnf * pg[...] + pbeta[...], 0.0)
    hid = dot(p, pw2[...]) + pb2[...]
    n = jnp.sqrt(jnp.sum(hid * hid, axis=1, keepdims=True))
    hid_ref[...] = hid / jnp.maximum(n, 1e-12)


def _tc_heads(fg, cw1, cb1, cg, cbeta, cw2, cb2, pw1, pb1, pg, pbeta, pw2, pb2):
    args = [fg, cw1, cb1.reshape(1, -1), cg.reshape(1, -1),
            cbeta.reshape(1, -1), cw2, cb2.reshape(1, -1),
            pw1, pb1.reshape(1, -1), pg.reshape(1, -1), pbeta.reshape(1, -1),
            pw2, pb2.reshape(1, -1)]
    return pl.pallas_call(
        _heads_body,
        in_specs=[pl.BlockSpec(a.shape, lambda: tuple(0 for _ in a.shape))
                  for a in args],
        out_specs=[pl.BlockSpec((BS, 3), lambda: (0, 0)),
                   pl.BlockSpec((BS, 128), lambda: (0, 0))],
        out_shape=[jax.ShapeDtypeStruct((BS, 3), jnp.float32),
                   jax.ShapeDtypeStruct((BS, 128), jnp.float32)],
    )(*args)


def kernel(vertices, dir0, w1, b1, dir1, w2, b2, dir2, w3, b3, dir3, w4, b4,
           dir4, cls_w1, cls_b1, cls_gamma, cls_beta, cls_w2, cls_b2,
           proj_w1, proj_b1, proj_gamma, proj_beta, proj_w2, proj_b2):
    idx17 = _knn_topk(vertices)
    nidx = idx17[:, :, 1:]
    gidx = (nidx + jnp.arange(BS, dtype=jnp.int32)[:, None, None] * V)
    gidx = gidx.reshape(-1)
    vflat = vertices.reshape(GV, 3)
    vx, vy, vz = vflat[:, 0], vflat[:, 1], vflat[:, 2]

    dxn, dyn, dzn, fm0f = _sc_ndn_fm0(vx, vy, vz, gidx, dir0.reshape(-1))
    fm0 = fm0f.reshape(GV, 32)

    c1, f1 = _tc_mm(fm0, None, w1, b1, 64)
    a1 = _sc_conv(gidx, dxn, dyn, dzn, f1, dir1.reshape(-1), 64)
    c2, f2 = _tc_mm(c1, a1.reshape(GV, 64), w2, b2, 128)
    a2 = _sc_conv(gidx, dxn, dyn, dzn, f2, dir2.reshape(-1), 128)
    c3, f3 = _tc_mm(c2, a2.reshape(GV, 128), w3, b3, 256)
    a3 = _sc_conv(gidx, dxn, dyn, dzn, f3, dir3.reshape(-1), 256)
    c4, f4 = _tc_mm(c3, a3.reshape(GV, 256), w4, b4, 1024)
    a4 = _sc_conv(gidx, dxn, dyn, dzn, f4, dir4.reshape(-1), 1024)

    fg = _tc_maxpool(c4, a4.reshape(GV, 1024))
    cls, hid = _tc_heads(fg, cls_w1, cls_b1, cls_gamma, cls_beta, cls_w2,
                         cls_b2, proj_w1, proj_b1, proj_gamma, proj_beta,
                         proj_w2, proj_b2)
    return (fg, cls, hid)


# speedup vs baseline: 9.2551x; 1.6753x over previous
import functools
import math

import jax
import jax.numpy as jnp
from jax import lax
from jax.experimental import pallas as pl
from jax.experimental.pallas import tpu as pltpu
from jax.experimental.pallas import tpu_sc as plsc

BS = 2
V = 2048
GV = BS * V
NB = 16
K1 = NB + 1
RB = 256

NC, NS, L = 2, 16, 16
NW = NC * NS
VID = GV // NW

_SC_MESH = dict(core_axis_name="c", subcore_axis_name="s")
_SC_PARAMS = pltpu.CompilerParams(needs_layout_passes=False,
                                  use_tc_tiling_on_sc=False)


def _topk_body(vb_ref, va_ref, idx_ref):
    vb = vb_ref[0]
    va = va_ref[0]
    inner = lax.dot_general(vb, va, (((1,), (1,)), ((), ())),
                            precision=lax.Precision.DEFAULT)
    qa = jnp.sum(va * va, axis=1)
    qb = jnp.sum(vb * vb, axis=1)
    dfull = (-2.0 * inner + qa[None, :]) + qb[:, None]
    H = RB // 2
    colsf = lax.broadcasted_iota(jnp.int32, (H, V), 1).astype(jnp.float32)
    VF = jnp.float32(V)
    d = [dfull[:H], dfull[H:]]
    picks = [[], []]
    for _ in range(K1):
        m = [jnp.min(d[h], axis=1, keepdims=True) for h in (0, 1)]
        cand = [jnp.where(d[h] == m[h], colsf, VF) for h in (0, 1)]
        jj = [jnp.min(cand[h], axis=1, keepdims=True) for h in (0, 1)]
        for h in (0, 1):
            picks[h].append(jj[h].astype(jnp.int32))
            d[h] = jnp.where(cand[h] == jj[h], jnp.inf, d[h])
    out = [jnp.concatenate(picks[h], axis=1) for h in (0, 1)]
    idx_ref[0] = jnp.concatenate(out, axis=0)


def _knn_topk(vertices):
    return pl.pallas_call(
        _topk_body,
        grid=(BS, V // RB),
        in_specs=[
            pl.BlockSpec((1, RB, 3), lambda b, r: (b, r, 0)),
            pl.BlockSpec((1, V, 3), lambda b, r: (b, 0, 0)),
        ],
        out_specs=pl.BlockSpec((1, RB, K1), lambda b, r: (b, r, 0)),
        out_shape=jax.ShapeDtypeStruct((BS, V, K1), jnp.int32),
    )(vertices, vertices)


def _rsqrt_nr(s):
    i = lax.bitcast_convert_type(s, jnp.int32)
    y = lax.bitcast_convert_type(
        jnp.int32(0x5F3759DF) - lax.shift_right_arithmetic(i, 1), jnp.float32)
    for _ in range(4):
        y = y * (1.5 - 0.5 * s * y * y)
    return y


def _inv_norm(s):
    return jnp.minimum(_rsqrt_nr(s), jnp.float32(1e12))


def _wid():
    return lax.axis_index("s") * NC + lax.axis_index("c")


def _sc_ndn_body(vx_h, vy_h, vz_h, gidx_h, d0_h,
                 dxn_h, dyn_h, dzn_h, fm0_h,
                 vxv, vyv, vzv, gidxv, d0v, dxnv, dynv, dznv, fm0v):
    wid = _wid()
    base = wid * VID
    pltpu.sync_copy(vx_h, vxv)
    pltpu.sync_copy(vy_h, vyv)
    pltpu.sync_copy(vz_h, vzv)
    pltpu.sync_copy(gidx_h.at[pl.ds(base * NB, VID * NB)], gidxv)
    pltpu.sync_copy(d0_h, d0v)

    sdn = []
    for c in range(2):
        sx = d0v[pl.ds(c * L, L)]
        sy = d0v[pl.ds(32 + c * L, L)]
        sz = d0v[pl.ds(64 + c * L, L)]
        r = _inv_norm(sx * sx + sy * sy + sz * sz)
        sdn.append((sx * r, sy * r, sz * r))

    def body(i, _):
        gv = gidxv[pl.ds(i * NB, NB)]
        ic = jnp.full((L,), base + i, jnp.int32)
        nx = plsc.load_gather(vxv, [gv])
        ny = plsc.load_gather(vyv, [gv])
        nz = plsc.load_gather(vzv, [gv])
        dx = nx - plsc.load_gather(vxv, [ic])
        dy = ny - plsc.load_gather(vyv, [ic])
        dz = nz - plsc.load_gather(vzv, [ic])
        r = _inv_norm(dx * dx + dy * dy + dz * dz)
        dxn = dx * r
        dyn = dy * r
        dzn = dz * r
        dxnv[pl.ds(i * NB, NB)] = dxn
        dynv[pl.ds(i * NB, NB)] = dyn
        dznv[pl.ds(i * NB, NB)] = dzn
        scal = [(dxn[n], dyn[n], dzn[n]) for n in range(NB)]
        for c in range(2):
            sx, sy, sz = sdn[c]
            acc = jnp.zeros((L,), jnp.float32)
            for n in range(NB):
                a, b_, c_ = scal[n]
                th = jnp.maximum(a * sx + b_ * sy + c_ * sz, 0.0)
                acc = jnp.maximum(acc, th)
            fm0v[pl.ds(i * 32 + c * L, L)] = acc
        return 0

    lax.fori_loop(0, VID, body, 0)

    pltpu.sync_copy(dxnv, dxn_h.at[pl.ds(base * NB, VID * NB)])
    pltpu.sync_copy(dynv, dyn_h.at[pl.ds(base * NB, VID * NB)])
    pltpu.sync_copy(dznv, dzn_h.at[pl.ds(base * NB, VID * NB)])
    pltpu.sync_copy(fm0v, fm0_h.at[pl.ds(base * 32, VID * 32)])


def _sc_ndn_fm0(vx, vy, vz, gidx, d0f):
    f = pl.kernel(
        _sc_ndn_body,
        out_type=(
            jax.ShapeDtypeStruct((GV * NB,), jnp.float32),
            jax.ShapeDtypeStruct((GV * NB,), jnp.float32),
            jax.ShapeDtypeStruct((GV * NB,), jnp.float32),
            jax.ShapeDtypeStruct((GV * 32,), jnp.float32),
        ),
        mesh=plsc.VectorSubcoreMesh(**_SC_MESH),
        compiler_params=_SC_PARAMS,
        scratch_types=[
            pltpu.VMEM((GV,), jnp.float32),
            pltpu.VMEM((GV,), jnp.float32),
            pltpu.VMEM((GV,), jnp.float32),
            pltpu.VMEM((VID * NB,), jnp.int32),
            pltpu.VMEM((96,), jnp.float32),
            pltpu.VMEM((VID * NB,), jnp.float32),
            pltpu.VMEM((VID * NB,), jnp.float32),
            pltpu.VMEM((VID * NB,), jnp.float32),
            pltpu.VMEM((VID * 32,), jnp.float32),
        ],
    )
    return f(vx, vy, vz, gidx, d0f)


def _sc_conv_body(co, gidx_h, dxn_h, dyn_h, dzn_h, f_h, dl_h, act_h,
                  gidxv, dxnv, dynv, dznv, dlv, snv,
                  fbuf0, fbuf1, actv0, actv1, semg0, semg1, sema0, sema1):
    wid = _wid()
    base = wid * VID
    pltpu.sync_copy(gidx_h.at[pl.ds(base * NB, VID * NB)], gidxv)
    pltpu.sync_copy(dxn_h.at[pl.ds(base * NB, VID * NB)], dxnv)
    pltpu.sync_copy(dyn_h.at[pl.ds(base * NB, VID * NB)], dynv)
    pltpu.sync_copy(dzn_h.at[pl.ds(base * NB, VID * NB)], dznv)
    pltpu.sync_copy(dl_h, dlv)

    nch = co // L

    def nrm(k, _):
        sx = dlv[pl.ds(k * L, L)]
        sy = dlv[pl.ds(co + k * L, L)]
        sz = dlv[pl.ds(2 * co + k * L, L)]
        r = _inv_norm(sx * sx + sy * sy + sz * sz)
        snv[pl.ds(k * L, L)] = sx * r
        snv[pl.ds(co + k * L, L)] = sy * r
        snv[pl.ds(2 * co + k * L, L)] = sz * r
        return 0

    lax.fori_loop(0, nch, nrm, 0)

    def issue_gather(i, fbuf, sem):
        gv = gidxv[pl.ds(i * NB, NB)]
        pltpu.async_copy(f_h.at[gv], fbuf, sem)

    def wait_gather(fbuf, sem):
        pltpu.make_async_copy(f_h.at[pl.ds(0, NB)], fbuf, sem).wait()

    def compute(i, fbuf, actv):
        dxv = dxnv[pl.ds(i * NB, NB)]
        dyv = dynv[pl.ds(i * NB, NB)]
        dzv = dznv[pl.ds(i * NB, NB)]
        scal = [(dxv[n], dyv[n], dzv[n]) for n in range(NB)]

        @plsc.parallel_loop(0, nch, 1, unroll=min(8, nch))
        def chunk(k):
            sx = snv[pl.ds(k * L, L)]
            sy = snv[pl.ds(co + k * L, L)]
            sz = snv[pl.ds(2 * co + k * L, L)]
            a, b_, c_ = scal[0]
            th = jnp.maximum(a * sx + b_ * sy + c_ * sz, 0.0)
            acc = th * fbuf[0, pl.ds(k * L, L)]
            for n in range(1, NB):
                a, b_, c_ = scal[n]
                th = jnp.maximum(a * sx + b_ * sy + c_ * sz, 0.0)
                acc = jnp.maximum(acc, th * fbuf[n, pl.ds(k * L, L)])
            actv[pl.ds(k * L, L)] = acc

    def issue_out(i, actv, sem):
        pltpu.async_copy(actv, act_h.at[pl.ds((base + i) * co, co)], sem)

    def wait_out(actv, sem):
        pltpu.make_async_copy(actv, act_h.at[pl.ds(base * co, co)], sem).wait()

    issue_gather(0, fbuf0, semg0)

    def body(i2, _):
        v0 = 2 * i2
        v1 = 2 * i2 + 1
        v2 = jnp.minimum(2 * i2 + 2, VID - 1)
        issue_gather(v1, fbuf1, semg1)
        wait_gather(fbuf0, semg0)

        @pl.when(i2 > 0)
        def _():
            wait_out(actv0, sema0)

        compute(v0, fbuf0, actv0)
        issue_out(v0, actv0, sema0)
        issue_gather(v2, fbuf0, semg0)
        wait_gather(fbuf1, semg1)

        @pl.when(i2 > 0)
        def _():
            wait_out(actv1, sema1)

        compute(v1, fbuf1, actv1)
        issue_out(v1, actv1, sema1)
        return 0

    lax.fori_loop(0, VID // 2, body, 0)
    wait_gather(fbuf0, semg0)
    wait_out(actv0, sema0)
    wait_out(actv1, sema1)


def _sc_conv(gidx, dxn, dyn, dzn, ftab, dlf, co):
    f = pl.kernel(
        functools.partial(_sc_conv_body, co),
        out_type=jax.ShapeDtypeStruct((GV * co,), jnp.float32),
        mesh=plsc.VectorSubcoreMesh(**_SC_MESH),
        compiler_params=_SC_PARAMS,
        scratch_types=[
            pltpu.VMEM((VID * NB,), jnp.int32),
            pltpu.VMEM((VID * NB,), jnp.float32),
            pltpu.VMEM((VID * NB,), jnp.float32),
            pltpu.VMEM((VID * NB,), jnp.float32),
            pltpu.VMEM((3 * co,), jnp.float32),
            pltpu.VMEM((3 * co,), jnp.float32),
            pltpu.VMEM((NB, co), jnp.float32),
            pltpu.VMEM((NB, co), jnp.float32),
            pltpu.VMEM((co,), jnp.float32),
            pltpu.VMEM((co,), jnp.float32),
            pltpu.SemaphoreType.DMA,
            pltpu.SemaphoreType.DMA,
            pltpu.SemaphoreType.DMA,
            pltpu.SemaphoreType.DMA,
        ],
    )
    return f(gidx, dxn, dyn, dzn, ftab, dlf)


def _mm_body(co, has_act, *refs):
    if has_act:
        c_ref, a_ref, w_ref, b_ref, cout_ref, f_ref = refs
        x = jnp.maximum(c_ref[...] + a_ref[...], 0.0)
    else:
        x_ref, w_ref, b_ref, cout_ref, f_ref = refs
        x = x_ref[...]
    out = lax.dot_general(x, w_ref[...], (((1,), (0,)), ((), ())),
                          precision=lax.Precision.DEFAULT) + b_ref[...]
    cout_ref[...] = out[:, :co]
    f_ref[...] = out[:, co:]


def _tc_mm(x_or_center, act, w, b, co):
    ci = w.shape[0]
    has_act = act is not None
    ins = ([x_or_center, act] if has_act else [x_or_center])
    ins += [w, b.reshape(1, 2 * co)]
    in_specs = [pl.BlockSpec((RB, ci), lambda r: (r, 0))
                for _ in range(2 if has_act else 1)]
    in_specs += [pl.BlockSpec((ci, 2 * co), lambda r: (0, 0)),
                 pl.BlockSpec((1, 2 * co), lambda r: (0, 0))]
    return pl.pallas_call(
        functools.partial(_mm_body, co, has_act),
        grid=(GV // RB,),
        in_specs=in_specs,
        out_specs=[pl.BlockSpec((RB, co), lambda r: (r, 0)),
                   pl.BlockSpec((RB, co), lambda r: (r, 0))],
        out_shape=[jax.ShapeDtypeStruct((GV, co), jnp.float32),
                   jax.ShapeDtypeStruct((GV, co), jnp.float32)],
    )(*ins)


def _maxpool_body(c_ref, a_ref, o_ref):
    r = pl.program_id(0)
    blk = c_ref[...] + a_ref[...]
    m = jnp.max(blk, axis=0, keepdims=True)

    @pl.when(r % (V // RB) == 0)
    def _():
        o_ref[0] = m

    @pl.when(r % (V // RB) != 0)
    def _():
        o_ref[0] = jnp.maximum(o_ref[0], m)


def _tc_maxpool(center, act):
    out = pl.pallas_call(
        _maxpool_body,
        grid=(GV // RB,),
        in_specs=[pl.BlockSpec((RB, 1024), lambda r: (r, 0)),
                  pl.BlockSpec((RB, 1024), lambda r: (r, 0))],
        out_specs=pl.BlockSpec((1, 1, 1024), lambda r: (r // (V // RB), 0, 0)),
        out_shape=jax.ShapeDtypeStruct((BS, 1, 1024), jnp.float32),
    )(center, act)
    return out.reshape(BS, 1024)


def _heads_body(fg_ref, cw1, cb1, cg, cbeta, cw2, cb2,
                pw1, pb1, pg, pbeta, pw2, pb2, cls_ref, hid_ref):
    fg = fg_ref[...]
    bnf = 1.0 / jnp.sqrt(jnp.float32(1.0 + 1e-5))
    dot = functools.partial(lax.dot_general,
                            dimension_numbers=(((1,), (0,)), ((), ())),
                            precision=lax.Precision.DEFAULT)
    h = dot(fg, cw1[...]) + cb1[...]
    h = jnp.maximum(h * bnf * cg[...] + cbeta[...], 0.0)
    cls_ref[...] = dot(h, cw2[...]) + cb2[...]
    p = dot(fg, pw1[...]) + pb1[...]
    p = jnp.maximum(p * bnf * pg[...] + pbeta[...], 0.0)
    hid = dot(p, pw2[...]) + pb2[...]
    n = jnp.sqrt(jnp.sum(hid * hid, axis=1, keepdims=True))
    hid_ref[...] = hid / jnp.maximum(n, 1e-12)


def _tc_heads(fg, cw1, cb1, cg, cbeta, cw2, cb2, pw1, pb1, pg, pbeta, pw2, pb2):
    args = [fg, cw1, cb1.reshape(1, -1), cg.reshape(1, -1),
            cbeta.reshape(1, -1), cw2, cb2.reshape(1, -1),
            pw1, pb1.reshape(1, -1), pg.reshape(1, -1), pbeta.reshape(1, -1),
            pw2, pb2.reshape(1, -1)]
    return pl.pallas_call(
        _heads_body,
        in_specs=[pl.BlockSpec(a.shape, lambda: tuple(0 for _ in a.shape))
                  for a in args],
        out_specs=[pl.BlockSpec((BS, 3), lambda: (0, 0)),
                   pl.BlockSpec((BS, 128), lambda: (0, 0))],
        out_shape=[jax.ShapeDtypeStruct((BS, 3), jnp.float32),
                   jax.ShapeDtypeStruct((BS, 128), jnp.float32)],
    )(*args)


def kernel(vertices, dir0, w1, b1, dir1, w2, b2, dir2, w3, b3, dir3, w4, b4,
           dir4, cls_w1, cls_b1, cls_gamma, cls_beta, cls_w2, cls_b2,
           proj_w1, proj_b1, proj_gamma, proj_beta, proj_w2, proj_b2):
    idx17 = _knn_topk(vertices)
    nidx = idx17[:, :, 1:]
    gidx = (nidx + jnp.arange(BS, dtype=jnp.int32)[:, None, None] * V)
    gidx = gidx.reshape(-1)
    vflat = vertices.reshape(GV, 3)
    vx, vy, vz = vflat[:, 0], vflat[:, 1], vflat[:, 2]

    dxn, dyn, dzn, fm0f = _sc_ndn_fm0(vx, vy, vz, gidx, dir0.reshape(-1))
    fm0 = fm0f.reshape(GV, 32)

    c1, f1 = _tc_mm(fm0, None, w1, b1, 64)
    a1 = _sc_conv(gidx, dxn, dyn, dzn, f1, dir1.reshape(-1), 64)
    c2, f2 = _tc_mm(c1, a1.reshape(GV, 64), w2, b2, 128)
    a2 = _sc_conv(gidx, dxn, dyn, dzn, f2, dir2.reshape(-1), 128)
    c3, f3 = _tc_mm(c2, a2.reshape(GV, 128), w3, b3, 256)
    a3 = _sc_conv(gidx, dxn, dyn, dzn, f3, dir3.reshape(-1), 256)
    c4, f4 = _tc_mm(c3, a3.reshape(GV, 256), w4, b4, 1024)
    a4 = _sc_conv(gidx, dxn, dyn, dzn, f4, dir4.reshape(-1), 1024)

    fg = _tc_maxpool(c4, a4.reshape(GV, 1024))
    cls, hid = _tc_heads(fg, cls_w1, cls_b1, cls_gamma, cls_beta, cls_w2,
                         cls_b2, proj_w1, proj_b1, proj_gamma, proj_beta,
                         proj_w2, proj_b2)
    return (fg, cls, hid)
